# baseline (device time: 36911 ns/iter reference)
import os

import jax
import jax.numpy as jnp
from jax import lax
from jax.experimental import pallas as pl
from jax.experimental.pallas import tpu as pltpu

_ABLATE = int(os.environ.get("KERNEL_ABLATE", "0"))

N_DEV = 32
NZ = 4
NY = 4
PLANE = 8
CH = 4
SUB = CH

X_OF = [0, 1, 1, 0, 0, 1, 1, 0]
Y_OF = [0, 0, 1, 1, 2, 2, 3, 3]
K_OF = {(X_OF[i], Y_OF[i]): i for i in range(PLANE)}
YUP = [K_OF[(X_OF[i], (Y_OF[i] + 1) % NY)] for i in range(PLANE)]
YDN = [K_OF[(X_OF[i], (Y_OF[i] + 3) % NY)] for i in range(PLANE)]


def _lut(idx, table):
    r = jnp.int32(table[0])
    for i in range(1, len(table)):
        r = jnp.where(idx == i, jnp.int32(table[i]), r)
    return r


def kernel(x, w_mat):
    m, _ = x.shape
    _, n = w_mat.shape
    mc = m // N_DEV
    nh = n // 2
    ws = nh // CH
    mp = m // NZ
    mq = mp // 4

    def body(x_ref, w_ref, out_ref, p_ref, ystage, sa, sb, sa2, sb2,
             a1s, a1r, a2s, a2r, a3s, a3r,
             b1s, b1r, b2s, b2r, b3s, b3r,
             a1ss, a1rs, a2ss, a2rs, a3ss, a3rs,
             b1ss, b1rs, b2ss, b2rs, b3ss, b3rs):
        my = lax.axis_index("i")
        j = my // PLANE
        k = lax.rem(my, PLANE)
        y = k // 2
        xc = lax.rem(k + y, 2)
        kx = k + 1 - 2 * lax.rem(k, 2)

        z_up = lax.rem(my + PLANE, N_DEV)
        z_dn = lax.rem(my + (N_DEV - PLANE), N_DEV)
        y_up = j * PLANE + _lut(k, YUP)
        y_dn = j * PLANE + _lut(k, YDN)
        x_pt = j * PLANE + kx

        if _ABLATE != 1:
            barrier_sem = pltpu.get_barrier_semaphore()
            for nbr in (z_dn, y_dn, x_pt):
                pl.semaphore_signal(
                    barrier_sem, inc=1,
                    device_id=(nbr,), device_id_type=pl.DeviceIdType.MESH,
                )

        p_ref[:, :] = jnp.dot(
            x_ref[:, :].astype(jnp.bfloat16),
            w_ref[:, :].astype(jnp.bfloat16),
            preferred_element_type=jnp.float32,
        )

        def restage():
            for yb in range(NY):
                for jp in range(NZ):
                    for xp in range(2):
                        ystage[pl.ds(mp * yb + mq * jp + mc * xp, mc), :] = (
                            p_ref[pl.ds(mc * (PLANE * jp + K_OF[(xp, yb)]),
                                        mc),
                                  pl.ds(nh, nh)]
                        )

        if _ABLATE == 1:
            restage()
            t1 = p_ref[pl.ds(my * mc, mc), pl.ds(0, nh)]
            t2 = ystage[pl.ds(y * mc, mc), :]
            out_ref[:, pl.ds(0, nh)] = t1 * jax.nn.sigmoid(t1)
            out_ref[:, pl.ds(nh, nh)] = t2 * jax.nn.sigmoid(t2)
            return

        pl.semaphore_wait(barrier_sem, 3)

        def make_chain(n_ticks, block, target, sbuf, rbuf, ssem, rsem, h):
            rds = [None] * n_ticks

            def start(s):
                val = block(s)
                if s > 0:
                    rds[s - 1].wait()
                    val = val + rbuf[h * n_ticks + s - 1, :, :]
                sbuf[h * n_ticks + s, :, :] = val
                r = pltpu.make_async_remote_copy(
                    src_ref=sbuf.at[h * n_ticks + s],
                    dst_ref=rbuf.at[h * n_ticks + s],
                    send_sem=ssem.at[h * n_ticks + s],
                    recv_sem=rsem.at[h * n_ticks + s],
                    device_id=(target,),
                    device_id_type=pl.DeviceIdType.MESH,
                )
                r.start()
                rds[s] = r

            def finish():
                rds[n_ticks - 1].wait()
                return rbuf[h * n_ticks + n_ticks - 1, :, :]

            return start, finish

        a1 = [
            make_chain(
                NZ - 1,
                lambda s, h=h: p_ref[
                    pl.ds(lax.rem(j + (NZ - 1 - s), NZ) * mp, mp),
                    pl.ds(h * ws, ws)],
                z_up, a1s, a1r, a1ss, a1rs, h,
            )
            for h in range(CH)
        ]
        b1 = [
            make_chain(
                NY - 1,
                lambda s, h=h: ystage[
                    pl.ds(lax.rem(y + (NY - 1 - s), NY) * mp, mp),
                    pl.ds(h * ws, ws)],
                y_up, b1s, b1r, b1ss, b1rs, h,
            )
            for h in range(CH)
        ]
        a2 = [
            make_chain(
                NY - 1,
                lambda s, h=h: sa[
                    pl.ds(lax.rem(y + (NY - 1 - s), NY) * mq, mq),
                    pl.ds(h * ws, ws)],
                y_up, a2s, a2r, a2ss, a2rs, h,
            )
            for h in range(CH)
        ]
        b2 = [
            make_chain(
                NZ - 1,
                lambda s, h=h: sb[
                    pl.ds(lax.rem(j + (NZ - 1 - s), NZ) * mq, mq),
                    pl.ds(h * ws, ws)],
                z_up, b2s, b2r, b2ss, b2rs, h,
            )
            for h in range(CH)
        ]

        for h in range(CH):
            a1[h][0](0)
        restage()
        for h in range(CH):
            b1[h][0](0)
        for s in range(1, 3):
            for h in range(CH):
                a1[h][0](s)
                b1[h][0](s)

        if _ABLATE == 2:
            for h in range(CH):
                sa[:, pl.ds(h * ws, ws)] = (
                    a1[h][1]() + p_ref[pl.ds(j * mp, mp), pl.ds(h * ws, ws)]
                )
                sb[:, pl.ds(h * ws, ws)] = (
                    b1[h][1]() + ystage[pl.ds(y * mp, mp), pl.ds(h * ws, ws)]
                )
            t1 = sa[pl.ds(k * mc, mc), :]
            t2 = sb[pl.ds(j * mc, mc), :]
            out_ref[:, pl.ds(0, nh)] = t1 * jax.nn.sigmoid(t1)
            out_ref[:, pl.ds(nh, nh)] = t2 * jax.nn.sigmoid(t2)
            return

        for h in range(CH):
            sa[:, pl.ds(h * ws, ws)] = (
                a1[h][1]() + p_ref[pl.ds(j * mp, mp), pl.ds(h * ws, ws)]
            )
            sb[:, pl.ds(h * ws, ws)] = (
                b1[h][1]() + ystage[pl.ds(y * mp, mp), pl.ds(h * ws, ws)]
            )
            a2[h][0](0)
            b2[h][0](0)
        for s in range(1, 3):
            for h in range(CH):
                a2[h][0](s)
                b2[h][0](s)

        kpar = lax.rem(k, 2)
        a3 = [
            make_chain(
                1,
                lambda s, h=h: sa2[pl.ds((1 - kpar) * mc, mc),
                                   pl.ds(h * ws, ws)],
                x_pt, a3s, a3r, a3ss, a3rs, h,
            )
            for h in range(CH)
        ]
        b3 = [
            make_chain(
                1,
                lambda s, h=h: sb2[pl.ds((1 - xc) * mc, mc),
                                   pl.ds(h * ws, ws)],
                x_pt, b3s, b3r, b3ss, b3rs, h,
            )
            for h in range(CH)
        ]
        for h in range(CH):
            sa2[:, pl.ds(h * ws, ws)] = (
                a2[h][1]() + sa[pl.ds(y * mq, mq), pl.ds(h * ws, ws)]
            )
            sb2[:, pl.ds(h * ws, ws)] = (
                b2[h][1]() + sb[pl.ds(j * mq, mq), pl.ds(h * ws, ws)]
            )
            a3[h][0](0)
            b3[h][0](0)
        for h in range(CH):
            ya = a3[h][1]() + sa2[pl.ds(kpar * mc, mc), pl.ds(h * ws, ws)]
            yb = b3[h][1]() + sb2[pl.ds(xc * mc, mc), pl.ds(h * ws, ws)]
            out_ref[:, pl.ds(h * ws, ws)] = ya * jax.nn.sigmoid(ya)
            out_ref[:, pl.ds(nh + h * ws, ws)] = yb * jax.nn.sigmoid(yb)

    f32 = jnp.float32
    return pl.pallas_call(
        body,
        out_shape=jax.ShapeDtypeStruct((mc, n), f32),
        in_specs=[
            pl.BlockSpec(memory_space=pltpu.VMEM),
            pl.BlockSpec(memory_space=pltpu.VMEM),
        ],
        out_specs=pl.BlockSpec(memory_space=pltpu.VMEM),
        scratch_shapes=[
            pltpu.VMEM((m, n), f32),
            pltpu.VMEM((m, nh), f32),
            pltpu.VMEM((mp, nh), f32),
            pltpu.VMEM((mp, nh), f32),
            pltpu.VMEM((mq, nh), f32),
            pltpu.VMEM((mq, nh), f32),
            pltpu.VMEM((CH * 3, mp, ws), f32),
            pltpu.VMEM((CH * 3, mp, ws), f32),
            pltpu.VMEM((CH * 3, mq, ws), f32),
            pltpu.VMEM((CH * 3, mq, ws), f32),
            pltpu.VMEM((CH, mc, ws), f32),
            pltpu.VMEM((CH, mc, ws), f32),
            pltpu.VMEM((CH * 3, mp, ws), f32),
            pltpu.VMEM((CH * 3, mp, ws), f32),
            pltpu.VMEM((CH * 3, mq, ws), f32),
            pltpu.VMEM((CH * 3, mq, ws), f32),
            pltpu.VMEM((CH, mc, ws), f32),
            pltpu.VMEM((CH, mc, ws), f32),
            pltpu.SemaphoreType.DMA((CH * 3,)),
            pltpu.SemaphoreType.DMA((CH * 3,)),
            pltpu.SemaphoreType.DMA((CH * 3,)),
            pltpu.SemaphoreType.DMA((CH * 3,)),
            pltpu.SemaphoreType.DMA((CH,)),
            pltpu.SemaphoreType.DMA((CH,)),
            pltpu.SemaphoreType.DMA((CH * 3,)),
            pltpu.SemaphoreType.DMA((CH * 3,)),
            pltpu.SemaphoreType.DMA((CH * 3,)),
            pltpu.SemaphoreType.DMA((CH * 3,)),
            pltpu.SemaphoreType.DMA((CH,)),
            pltpu.SemaphoreType.DMA((CH,)),
        ],
        compiler_params=pltpu.CompilerParams(
            collective_id=None if _ABLATE == 1 else 0
        ),
    )(x, w_mat)


# device time: 33892 ns/iter; 1.0891x vs baseline; 1.0891x over previous
import contextlib
import os

import jax
import jax.numpy as jnp
from jax import lax
from jax.experimental import pallas as pl
from jax.experimental.pallas import tpu as pltpu

_ABLATE = int(os.environ.get("KERNEL_ABLATE", "0"))
_SCOPES = int(os.environ.get("KERNEL_SCOPES", "0"))


def _scope(name):
    return jax.named_scope(name) if _SCOPES else contextlib.nullcontext()

N_DEV = 32
NZ = 4
NY = 4
PLANE = 8
CH = 4
SUB = CH

X_OF = [0, 1, 1, 0, 0, 1, 1, 0]
Y_OF = [0, 0, 1, 1, 2, 2, 3, 3]
K_OF = {(X_OF[i], Y_OF[i]): i for i in range(PLANE)}
YUP = [K_OF[(X_OF[i], (Y_OF[i] + 1) % NY)] for i in range(PLANE)]
Y2 = [K_OF[(X_OF[i], (Y_OF[i] + 2) % NY)] for i in range(PLANE)]
YDN = [K_OF[(X_OF[i], (Y_OF[i] + 3) % NY)] for i in range(PLANE)]


def _lut(idx, table):
    r = jnp.int32(table[0])
    for i in range(1, len(table)):
        r = jnp.where(idx == i, jnp.int32(table[i]), r)
    return r


def kernel(x, w_mat):
    m, _ = x.shape
    _, n = w_mat.shape
    mc = m // N_DEV
    nh = n // 2
    ws = nh // CH
    mp = m // NZ
    mq = mp // 4

    def body(x_ref, w_ref, out_ref, p_ref, ystage, sa, sb, sa2, sb2,
             a1s, a1r, a2s, a2r, a3s, a3r,
             b1s, b1r, b2s, b2r, b3s, b3r,
             a1ss, a1rs, a2ss, a2rs, a3ss, a3rs,
             b1ss, b1rs, b2ss, b2rs, b3ss, b3rs):
        my = lax.axis_index("i")
        j = my // PLANE
        k = lax.rem(my, PLANE)
        y = k // 2
        xc = lax.rem(k + y, 2)
        kx = k + 1 - 2 * lax.rem(k, 2)

        z_up = lax.rem(my + PLANE, N_DEV)
        z_dn = lax.rem(my + (N_DEV - PLANE), N_DEV)
        z_2 = lax.rem(my + 2 * PLANE, N_DEV)
        y_up = j * PLANE + _lut(k, YUP)
        y_2 = j * PLANE + _lut(k, Y2)
        y_dn = j * PLANE + _lut(k, YDN)
        x_pt = j * PLANE + kx

        if _ABLATE != 1:
            barrier_sem = pltpu.get_barrier_semaphore()
            for nbr in (z_dn, y_dn, x_pt):
                pl.semaphore_signal(
                    barrier_sem, inc=1,
                    device_id=(nbr,), device_id_type=pl.DeviceIdType.MESH,
                )

        with _scope("dot"):
            p_ref[:, :] = jnp.dot(
                x_ref[:, :].astype(jnp.bfloat16),
                w_ref[:, :].astype(jnp.bfloat16),
                preferred_element_type=jnp.float32,
            )

        def restage():
            for yb in range(NY):
                for jp in range(NZ):
                    for xp in range(2):
                        ystage[pl.ds(mp * yb + mq * jp + mc * xp, mc), :] = (
                            p_ref[pl.ds(mc * (PLANE * jp + K_OF[(xp, yb)]),
                                        mc),
                                  pl.ds(nh, nh)]
                        )

        if _ABLATE == 1:
            restage()
            t1 = p_ref[pl.ds(my * mc, mc), pl.ds(0, nh)]
            t2 = ystage[pl.ds(y * mc, mc), :]
            out_ref[:, pl.ds(0, nh)] = t1 * jax.nn.sigmoid(t1)
            out_ref[:, pl.ds(nh, nh)] = t2 * jax.nn.sigmoid(t2)
            return

        with _scope("barrier"):
            pl.semaphore_wait(barrier_sem, 3)

        def make_chain(n_ticks, block, target, sbuf, rbuf, ssem, rsem, h):
            rds = [None] * n_ticks

            def start(s):
                val = block(s)
                if s > 0:
                    rds[s - 1].wait()
                    val = val + rbuf[h * n_ticks + s - 1, :, :]
                sbuf[h * n_ticks + s, :, :] = val
                r = pltpu.make_async_remote_copy(
                    src_ref=sbuf.at[h * n_ticks + s],
                    dst_ref=rbuf.at[h * n_ticks + s],
                    send_sem=ssem.at[h * n_ticks + s],
                    recv_sem=rsem.at[h * n_ticks + s],
                    device_id=(target,),
                    device_id_type=pl.DeviceIdType.MESH,
                )
                r.start()
                rds[s] = r

            def finish():
                rds[n_ticks - 1].wait()
                return rbuf[h * n_ticks + n_ticks - 1, :, :]

            return start, finish

        a1 = [
            make_chain(
                NZ - 1,
                lambda s, h=h: p_ref[
                    pl.ds(lax.rem(j + (NZ - 1 - s), NZ) * mp, mp),
                    pl.ds(h * ws, ws)],
                z_up, a1s, a1r, a1ss, a1rs, h,
            )
            for h in range(CH)
        ]
        b1 = [
            make_chain(
                NY - 1,
                lambda s, h=h: ystage[
                    pl.ds(lax.rem(y + (NY - 1 - s), NY) * mp, mp),
                    pl.ds(h * ws, ws)],
                y_up, b1s, b1r, b1ss, b1rs, h,
            )
            for h in range(CH)
        ]
        a2rd = [None] * (CH * 3)
        b2rd = [None] * (CH * 3)
        y_at = [y_up, y_2, y_dn]
        z_at = [z_up, z_2, z_dn]

        def p2_start(h):
            for d in (1, 2, 3):
                slot = h * 3 + d - 1
                a2s[slot, :, :] = sa[pl.ds(lax.rem(y + d, NY) * mq, mq),
                                     pl.ds(h * ws, ws)]
                r = pltpu.make_async_remote_copy(
                    src_ref=a2s.at[slot],
                    dst_ref=a2r.at[slot],
                    send_sem=a2ss.at[slot],
                    recv_sem=a2rs.at[slot],
                    device_id=(y_at[d - 1],),
                    device_id_type=pl.DeviceIdType.MESH,
                )
                r.start()
                a2rd[slot] = r
                b2s[slot, :, :] = sb[pl.ds(lax.rem(j + d, NZ) * mq, mq),
                                     pl.ds(h * ws, ws)]
                r = pltpu.make_async_remote_copy(
                    src_ref=b2s.at[slot],
                    dst_ref=b2r.at[slot],
                    send_sem=b2ss.at[slot],
                    recv_sem=b2rs.at[slot],
                    device_id=(z_at[d - 1],),
                    device_id_type=pl.DeviceIdType.MESH,
                )
                r.start()
                b2rd[slot] = r

        def p2_finish(h):
            acc_a = sa[pl.ds(y * mq, mq), pl.ds(h * ws, ws)]
            acc_b = sb[pl.ds(j * mq, mq), pl.ds(h * ws, ws)]
            for d in (1, 2, 3):
                slot = h * 3 + d - 1
                a2rd[slot].wait()
                acc_a = acc_a + a2r[slot, :, :]
                b2rd[slot].wait()
                acc_b = acc_b + b2r[slot, :, :]
            sa2[:, pl.ds(h * ws, ws)] = acc_a
            sb2[:, pl.ds(h * ws, ws)] = acc_b

        with _scope("p1_t0a"):
            for h in range(CH):
                a1[h][0](0)
        with _scope("restage"):
            restage()
        with _scope("p1_t0b"):
            for h in range(CH):
                b1[h][0](0)
        for s in range(1, 3):
            with _scope(f"p1_row{s}"):
                for h in range(CH):
                    a1[h][0](s)
                    b1[h][0](s)

        if _ABLATE == 2:
            for h in range(CH):
                sa[:, pl.ds(h * ws, ws)] = (
                    a1[h][1]() + p_ref[pl.ds(j * mp, mp), pl.ds(h * ws, ws)]
                )
                sb[:, pl.ds(h * ws, ws)] = (
                    b1[h][1]() + ystage[pl.ds(y * mp, mp), pl.ds(h * ws, ws)]
                )
            t1 = sa[pl.ds(k * mc, mc), :]
            t2 = sb[pl.ds(j * mc, mc), :]
            out_ref[:, pl.ds(0, nh)] = t1 * jax.nn.sigmoid(t1)
            out_ref[:, pl.ds(nh, nh)] = t2 * jax.nn.sigmoid(t2)
            return

        for h in range(CH):
            with _scope(f"p1fin_p2t0_{h}"):
                sa[:, pl.ds(h * ws, ws)] = (
                    a1[h][1]() + p_ref[pl.ds(j * mp, mp), pl.ds(h * ws, ws)]
                )
                sb[:, pl.ds(h * ws, ws)] = (
                    b1[h][1]() + ystage[pl.ds(y * mp, mp), pl.ds(h * ws, ws)]
                )
                p2_start(h)

        kpar = lax.rem(k, 2)
        a3 = [
            make_chain(
                1,
                lambda s, h=h: sa2[pl.ds((1 - kpar) * mc, mc),
                                   pl.ds(h * ws, ws)],
                x_pt, a3s, a3r, a3ss, a3rs, h,
            )
            for h in range(CH)
        ]
        b3 = [
            make_chain(
                1,
                lambda s, h=h: sb2[pl.ds((1 - xc) * mc, mc),
                                   pl.ds(h * ws, ws)],
                x_pt, b3s, b3r, b3ss, b3rs, h,
            )
            for h in range(CH)
        ]
        for h in range(CH):
            with _scope(f"p2fin_p3_{h}"):
                p2_finish(h)
                a3[h][0](0)
                b3[h][0](0)
        for h in range(CH):
            with _scope(f"tail_{h}"):
                ya = a3[h][1]() + sa2[pl.ds(kpar * mc, mc),
                                      pl.ds(h * ws, ws)]
                yb = b3[h][1]() + sb2[pl.ds(xc * mc, mc), pl.ds(h * ws, ws)]
                out_ref[:, pl.ds(h * ws, ws)] = ya * jax.nn.sigmoid(ya)
                out_ref[:, pl.ds(nh + h * ws, ws)] = yb * jax.nn.sigmoid(yb)

    f32 = jnp.float32
    return pl.pallas_call(
        body,
        out_shape=jax.ShapeDtypeStruct((mc, n), f32),
        in_specs=[
            pl.BlockSpec(memory_space=pltpu.VMEM),
            pl.BlockSpec(memory_space=pltpu.VMEM),
        ],
        out_specs=pl.BlockSpec(memory_space=pltpu.VMEM),
        scratch_shapes=[
            pltpu.VMEM((m, n), f32),
            pltpu.VMEM((m, nh), f32),
            pltpu.VMEM((mp, nh), f32),
            pltpu.VMEM((mp, nh), f32),
            pltpu.VMEM((mq, nh), f32),
            pltpu.VMEM((mq, nh), f32),
            pltpu.VMEM((CH * 3, mp, ws), f32),
            pltpu.VMEM((CH * 3, mp, ws), f32),
            pltpu.VMEM((CH * 3, mq, ws), f32),
            pltpu.VMEM((CH * 3, mq, ws), f32),
            pltpu.VMEM((CH, mc, ws), f32),
            pltpu.VMEM((CH, mc, ws), f32),
            pltpu.VMEM((CH * 3, mp, ws), f32),
            pltpu.VMEM((CH * 3, mp, ws), f32),
            pltpu.VMEM((CH * 3, mq, ws), f32),
            pltpu.VMEM((CH * 3, mq, ws), f32),
            pltpu.VMEM((CH, mc, ws), f32),
            pltpu.VMEM((CH, mc, ws), f32),
            pltpu.SemaphoreType.DMA((CH * 3,)),
            pltpu.SemaphoreType.DMA((CH * 3,)),
            pltpu.SemaphoreType.DMA((CH * 3,)),
            pltpu.SemaphoreType.DMA((CH * 3,)),
            pltpu.SemaphoreType.DMA((CH,)),
            pltpu.SemaphoreType.DMA((CH,)),
            pltpu.SemaphoreType.DMA((CH * 3,)),
            pltpu.SemaphoreType.DMA((CH * 3,)),
            pltpu.SemaphoreType.DMA((CH * 3,)),
            pltpu.SemaphoreType.DMA((CH * 3,)),
            pltpu.SemaphoreType.DMA((CH,)),
            pltpu.SemaphoreType.DMA((CH,)),
        ],
        compiler_params=pltpu.CompilerParams(
            collective_id=None if _ABLATE == 1 else 0
        ),
    )(x, w_mat)
